# baseline (device time: 20798 ns/iter reference)
import os

import jax
import jax.numpy as jnp
from jax import lax
from jax.experimental import pallas as pl
from jax.experimental.pallas import tpu as pltpu

N_DEV = 32
BLK = 64
N_Q = int(os.environ.get("KERNEL_NQ", "4"))
PER_Q = N_DEV // N_Q
LOG2_N = 5

_ABLATE = set(os.environ.get("KERNEL_ABLATE", "").split(","))

_MS = getattr(pltpu, "MemorySpace", None) or getattr(pltpu, "TPUMemorySpace")
_ANY = getattr(_MS, "HBM", None) or getattr(pl, "ANY")
_sem_signal = getattr(pl, "semaphore_signal", None) or pltpu.semaphore_signal
_sem_wait = getattr(pl, "semaphore_wait", None) or pltpu.semaphore_wait
_DevIdType = getattr(pl, "DeviceIdType", None) or pltpu.DeviceIdType


def kernel(x, w_mat):
    k_global, k_shard = x.shape
    _, n = w_mat.shape
    assert k_shard == BLK and k_global == N_DEV * BLK
    kq = k_global // N_Q

    def body(x_ref, w_hbm, out_ref, xbf_ref, stack_ref, xg_ref, wf_ref,
             acc_ref, send_sem, recv_qsems, wdma_sems, round_sems):
        my = lax.axis_index("i")
        myq = my // PER_Q

        barrier_sem = pltpu.get_barrier_semaphore()
        _sem_signal(barrier_sem, 1)
        _sem_wait(barrier_sem, 1)

        def wdma(q):
            return pltpu.make_async_copy(
                w_hbm.at[pl.ds(q * kq, kq), :],
                wf_ref.at[pl.ds(q * kq, kq), :],
                wdma_sems.at[q],
            )

        for q in range(N_Q):
            wdma(q).start()

        xbf_ref[...] = x_ref[...].astype(jnp.bfloat16)

        def send_desc(k):
            tgt = lax.rem(my + k, N_DEV)
            return pltpu.make_async_remote_copy(
                src_ref=xbf_ref.at[pl.ds(tgt * BLK, BLK), :],
                dst_ref=stack_ref.at[my],
                send_sem=send_sem,
                recv_sem=recv_qsems.at[myq],
                device_id=(tgt,),
                device_id_type=_DevIdType.MESH,
            )

        if "nosends" not in _ABLATE:
            for k in range(N_DEV):
                send_desc(k).start()

        for q in range(N_Q):
            if "nosends" not in _ABLATE:
                qsl = stack_ref.at[pl.ds(q * PER_Q, PER_Q)]
                pltpu.make_async_remote_copy(
                    src_ref=qsl,
                    dst_ref=qsl,
                    send_sem=send_sem,
                    recv_sem=recv_qsems.at[q],
                    device_id=(0,),
                    device_id_type=_DevIdType.MESH,
                ).wait_recv()
            if "noassemble" not in _ABLATE:
                for j in range(q * PER_Q, (q + 1) * PER_Q):
                    xg_ref[:, j * BLK:(j + 1) * BLK] = stack_ref[j]
            wdma(q).wait()
            if "nogemm" in _ABLATE:
                if q == N_Q - 1:
                    out_ref[...] = jnp.zeros_like(out_ref)
                continue
            yq = jnp.dot(
                xg_ref[:, q * kq:(q + 1) * kq],
                wf_ref[q * kq:(q + 1) * kq, :].astype(jnp.bfloat16),
                preferred_element_type=jnp.float32,
            )
            if q == 0:
                acc_ref[...] = yq
            elif q < N_Q - 1:
                acc_ref[...] += yq
            else:
                out_ref[...] = jnp.maximum(acc_ref[...] + yq, 0.0)

        if "nosends" not in _ABLATE:
            pltpu.make_async_remote_copy(
                src_ref=xbf_ref,
                dst_ref=xbf_ref,
                send_sem=send_sem,
                recv_sem=recv_qsems.at[0],
                device_id=(0,),
                device_id_type=_DevIdType.MESH,
            ).wait_send()

        if "nobarrier" not in _ABLATE:
            for r in range(LOG2_N):
                _sem_signal(round_sems.at[r], 1,
                            device_id=(lax.rem(my + (1 << r), N_DEV),),
                            device_id_type=_DevIdType.MESH)
                _sem_wait(round_sems.at[r], 1)

    return pl.pallas_call(
        body,
        out_shape=jax.ShapeDtypeStruct((BLK, n), jnp.float32),
        in_specs=[
            pl.BlockSpec(memory_space=pltpu.VMEM),
            pl.BlockSpec(memory_space=_ANY),
        ],
        out_specs=pl.BlockSpec(memory_space=pltpu.VMEM),
        scratch_shapes=[
            pltpu.VMEM((k_global, BLK), jnp.bfloat16),
            pltpu.VMEM((N_DEV, BLK, BLK), jnp.bfloat16),
            pltpu.VMEM((BLK, k_global), jnp.bfloat16),
            pltpu.VMEM((k_global, n), jnp.float32),
            pltpu.VMEM((BLK, n), jnp.float32),
            pltpu.SemaphoreType.DMA,
            pltpu.SemaphoreType.DMA((N_Q,)),
            pltpu.SemaphoreType.DMA((N_Q,)),
            pltpu.SemaphoreType.REGULAR((LOG2_N,)),
        ],
        compiler_params=pltpu.CompilerParams(
            collective_id=0,
            disable_bounds_checks="boundschecks" not in _ABLATE,
        ),
    )(x, pltpu.with_memory_space_constraint(w_mat, _ANY))
